# baseline (speedup 1.0000x reference)
import numpy as np
import jax
import jax.numpy as jnp
from jax import lax
from jax.experimental import pallas as pl
from jax.experimental.pallas import tpu as pltpu
from jax.experimental.pallas import tpu_sc as plsc

NUM_TERMS = 16
NUM_COLS = 16384
DEPTH = 128
LANES = 16
VECS_PER_ROW = DEPTH // LANES

TC_COLS = 2048
TC_BLK = 512
SC_COLS = NUM_COLS - TC_COLS
NUM_WORKERS = 32
COLS_PER_W = SC_COLS // NUM_WORKERS
PAD_PER_W = 512
CHUNK_LENS = (128, 128, 128, COLS_PER_W - 384)
NUM_CHUNKS = len(CHUNK_LENS)


def _sc_body(xp_hbm, pe_hbm, out_hbm,
             idx0, idx1, idx2, idx3, acc0, acc1, acc2, acc3, stage0, stage1,
             gsem0, gsem1, gsem2, gsem3, wsem0, wsem1):
    cid = lax.axis_index("c")
    sid = lax.axis_index("s")
    wid = sid * 2 + cid
    xcol0 = wid * PAD_PER_W
    ocol0 = wid * COLS_PER_W
    inv = jnp.float32(1.0 / NUM_TERMS)

    idxs = (idx0, idx1, idx2, idx3)
    accs = (acc0, acc1, acc2, acc3)
    stages = (stage0, stage1)
    gsems = (gsem0, gsem1, gsem2, gsem3)
    wsems = (wsem0, wsem1)

    def stage_idx(k):
        pltpu.sync_copy(xp_hbm.at[:, pl.ds(xcol0 + k * 128, 128)], idxs[k])

    def gather(k, i, add):
        return pltpu.async_copy(
            pe_hbm.at[idxs[k].at[i, pl.ds(0, CHUNK_LENS[k])]],
            accs[k].at[pl.ds(0, CHUNK_LENS[k])], gsems[k], add=add)

    stage_idx(0)
    first = {0: gather(0, 0, False)}
    for k in range(1, NUM_CHUNKS):
        stage_idx(k)
        first[k] = gather(k, 0, False)
    pending = {}
    for k in range(NUM_CHUNKS):
        first.pop(k).wait()
        pending[k] = [gather(k, i, True) for i in range(1, NUM_TERMS)]

    wb = {}
    for k in range(NUM_CHUNKS):
        acc, stage, cl = accs[k], stages[k % 2], CHUNK_LENS[k]
        for cd in pending.pop(k):
            cd.wait()
        if k - 2 in wb:
            wb.pop(k - 2).wait()

        def row_body(r2, carry):
            for r in (2 * r2, 2 * r2 + 1):
                for j in range(VECS_PER_ROW):
                    sl = pl.ds(j * LANES, LANES)
                    stage[r, sl] = acc[r, sl] * inv
            return carry

        lax.fori_loop(0, cl // 2, row_body, 0)
        wb[k] = pltpu.async_copy(
            stage.at[pl.ds(0, cl)],
            out_hbm.at[pl.ds(ocol0 + k * 128, cl)],
            wsems[k % 2])
    for k in sorted(wb):
        wb.pop(k).wait()


def _make_fp():
    d = np.arange(DEPTH)
    f = np.exp((d // 2) * 2 * (-3.5 / DEPTH)).astype(np.float32)
    p = ((d % 2) * (np.pi / 2)).astype(np.float32)
    return np.stack([f, p]).astype(np.float32)


_FP = _make_fp()


def _tc_body(xt_ref, fp_ref, out_ref):
    f = fp_ref[0:1, :]
    p = fp_ref[1:2, :]
    acc = jnp.zeros((TC_BLK, DEPTH), jnp.float32)
    for i in range(NUM_TERMS):
        xi = xt_ref[:, i:i + 1].astype(jnp.float32)
        acc = acc + jnp.sin(xi * f + p)
    out_ref[:, :] = acc * jnp.float32(1.0 / NUM_TERMS)


@jax.jit
def kernel(x, position_encoding):
    xs = x[:, TC_COLS:].reshape(NUM_TERMS, NUM_WORKERS, COLS_PER_W)
    xp = jnp.pad(xs, ((0, 0), (0, 0), (0, PAD_PER_W - COLS_PER_W)))
    xp = xp.reshape(NUM_TERMS, NUM_WORKERS * PAD_PER_W)

    mesh = plsc.VectorSubcoreMesh(core_axis_name="c", subcore_axis_name="s")
    sc_call = pl.kernel(
        _sc_body,
        mesh=mesh,
        out_type=jax.ShapeDtypeStruct((SC_COLS, DEPTH), jnp.float32),
        scratch_types=(
            [pltpu.VMEM((NUM_TERMS, 128), jnp.int32)] * 4
            + [pltpu.VMEM((128, DEPTH), jnp.float32)] * 6
            + [pltpu.SemaphoreType.DMA] * 6
        ),
    )
    sc_out = sc_call(xp, position_encoding)

    xt = x[:, :TC_COLS].T
    fp = jnp.asarray(_FP)
    tc_out = pl.pallas_call(
        _tc_body,
        grid=(TC_COLS // TC_BLK,),
        in_specs=[
            pl.BlockSpec((TC_BLK, NUM_TERMS), lambda b: (b, 0)),
            pl.BlockSpec((2, DEPTH), lambda b: (0, 0)),
        ],
        out_specs=pl.BlockSpec((TC_BLK, DEPTH), lambda b: (b, 0)),
        out_shape=jax.ShapeDtypeStruct((TC_COLS, DEPTH), jnp.float32),
    )(xt, fp)

    return jnp.concatenate([tc_out, sc_out], axis=0)


# speedup vs baseline: 1.1806x; 1.1806x over previous
import jax
import jax.numpy as jnp
from jax import lax
from jax.experimental import pallas as pl
from jax.experimental.pallas import tpu as pltpu
from jax.experimental.pallas import tpu_sc as plsc

NUM_TERMS = 16
NUM_COLS = 16384
DEPTH = 128
NUM_WORKERS = 32
COLS_PER_W = NUM_COLS // NUM_WORKERS
CHUNK = 128
NUM_CHUNKS = COLS_PER_W // CHUNK
LANES = 16
VECS_PER_ROW = DEPTH // LANES


def _body(x_hbm, pe_hbm, out_hbm,
          idx_v, acc0, acc1, acc2, acc3, stage0, stage1,
          gsem0, gsem1, gsem2, gsem3, wsem0, wsem1):
    cid = lax.axis_index("c")
    sid = lax.axis_index("s")
    wid = sid * 2 + cid
    col0 = wid * COLS_PER_W
    inv = jnp.float32(1.0 / NUM_TERMS)
    zvec = jnp.zeros((LANES,), jnp.float32)

    accs = (acc0, acc1, acc2, acc3)
    stages = (stage0, stage1)
    gsems = (gsem0, gsem1, gsem2, gsem3)
    wsems = (wsem0, wsem1)

    def zero_acc(acc):
        def zbody(r, carry):
            for j in range(VECS_PER_ROW):
                acc[r, pl.ds(j * LANES, LANES)] = zvec
            return carry
        lax.fori_loop(0, CHUNK, zbody, 0)

    def fire(k):
        return [
            pltpu.async_copy(
                pe_hbm.at[idx_v.at[i, pl.ds(k * CHUNK, CHUNK)]],
                accs[k], gsems[k], add=True)
            for i in range(NUM_TERMS)
        ]

    zero_acc(acc0)
    pltpu.sync_copy(x_hbm.at[:, pl.ds(col0, CHUNK)], idx_v.at[:, pl.ds(0, CHUNK)])
    pending = {0: fire(0)}
    pltpu.sync_copy(x_hbm.at[:, pl.ds(col0 + CHUNK, COLS_PER_W - CHUNK)],
                    idx_v.at[:, pl.ds(CHUNK, COLS_PER_W - CHUNK)])
    for k in range(1, NUM_CHUNKS):
        zero_acc(accs[k])
        pending[k] = fire(k)

    wb = {}
    for k in range(NUM_CHUNKS):
        acc, stage = accs[k], stages[k % 2]
        for cd in pending.pop(k):
            cd.wait()
        if k - 2 in wb:
            wb.pop(k - 2).wait()

        def row_body(r2, carry):
            for r in (2 * r2, 2 * r2 + 1):
                for j in range(VECS_PER_ROW):
                    sl = pl.ds(j * LANES, LANES)
                    stage[r, sl] = acc[r, sl] * inv
            return carry

        lax.fori_loop(0, CHUNK // 2, row_body, 0)
        wb[k] = pltpu.async_copy(
            stage, out_hbm.at[pl.ds(col0 + k * CHUNK, CHUNK)], wsems[k % 2])
    for k in sorted(wb):
        wb.pop(k).wait()


@jax.jit
def kernel(x, position_encoding):
    mesh = plsc.VectorSubcoreMesh(core_axis_name="c", subcore_axis_name="s")
    f = pl.kernel(
        _body,
        mesh=mesh,
        out_type=jax.ShapeDtypeStruct((NUM_COLS, DEPTH), jnp.float32),
        scratch_types=(
            [pltpu.VMEM((NUM_TERMS, COLS_PER_W), jnp.int32)]
            + [pltpu.VMEM((CHUNK, DEPTH), jnp.float32)] * 6
            + [pltpu.SemaphoreType.DMA] * 6
        ),
    )
    return f(x, position_encoding)


# speedup vs baseline: 1.1847x; 1.0035x over previous
import jax
import jax.numpy as jnp
from jax import lax
from jax.experimental import pallas as pl
from jax.experimental.pallas import tpu as pltpu
from jax.experimental.pallas import tpu_sc as plsc

NUM_TERMS = 16
NUM_COLS = 16384
DEPTH = 128
NUM_WORKERS = 32
COLS_PER_W = NUM_COLS // NUM_WORKERS
CHUNK = 128
NUM_CHUNKS = COLS_PER_W // CHUNK
LANES = 16
VECS_PER_ROW = DEPTH // LANES


def _body(x_hbm, pe_hbm, out_hbm,
          idx_v, acc0, acc1, acc2, acc3,
          gsem0, gsem1, gsem2, gsem3, wsem0):
    cid = lax.axis_index("c")
    sid = lax.axis_index("s")
    wid = sid * 2 + cid
    col0 = wid * COLS_PER_W
    inv = jnp.float32(1.0 / NUM_TERMS)
    zvec = jnp.zeros((LANES,), jnp.float32)

    accs = (acc0, acc1, acc2, acc3)
    gsems = (gsem0, gsem1, gsem2, gsem3)
    wsems = (wsem0,)

    def zero_acc(acc):
        def zbody(r, carry):
            for j in range(VECS_PER_ROW):
                acc[r, pl.ds(j * LANES, LANES)] = zvec
            return carry
        lax.fori_loop(0, CHUNK, zbody, 0)

    def fire(k):
        return [
            pltpu.async_copy(
                pe_hbm.at[idx_v.at[i, pl.ds(k * CHUNK, CHUNK)]],
                accs[k], gsems[k], add=True)
            for i in range(NUM_TERMS)
        ]

    zero_acc(acc0)
    pltpu.sync_copy(x_hbm.at[:, pl.ds(col0, CHUNK)], idx_v.at[:, pl.ds(0, CHUNK)])
    pending = {0: fire(0)}
    pltpu.sync_copy(x_hbm.at[:, pl.ds(col0 + CHUNK, COLS_PER_W - CHUNK)],
                    idx_v.at[:, pl.ds(CHUNK, COLS_PER_W - CHUNK)])
    for k in range(1, NUM_CHUNKS):
        zero_acc(accs[k])
        pending[k] = fire(k)

    wb = []
    for k in range(NUM_CHUNKS):
        acc = accs[k]
        for cd in pending.pop(k):
            cd.wait()

        def row_body(r2, carry):
            for r in (2 * r2, 2 * r2 + 1):
                for j in range(VECS_PER_ROW):
                    sl = pl.ds(j * LANES, LANES)
                    acc[r, sl] = acc[r, sl] * inv
            return carry

        lax.fori_loop(0, CHUNK // 2, row_body, 0)
        wb.append(pltpu.async_copy(
            acc, out_hbm.at[pl.ds(col0 + k * CHUNK, CHUNK)], wsems[0]))
    for cd in wb:
        cd.wait()


@jax.jit
def kernel(x, position_encoding):
    mesh = plsc.VectorSubcoreMesh(core_axis_name="c", subcore_axis_name="s")
    f = pl.kernel(
        _body,
        mesh=mesh,
        out_type=jax.ShapeDtypeStruct((NUM_COLS, DEPTH), jnp.float32),
        scratch_types=(
            [pltpu.VMEM((NUM_TERMS, COLS_PER_W), jnp.int32)]
            + [pltpu.VMEM((CHUNK, DEPTH), jnp.float32)] * 4
            + [pltpu.SemaphoreType.DMA] * 5
        ),
    )
    return f(x, position_encoding)


# speedup vs baseline: 1.1884x; 1.0031x over previous
import jax
import jax.numpy as jnp
from jax import lax
from jax.experimental import pallas as pl
from jax.experimental.pallas import tpu as pltpu
from jax.experimental.pallas import tpu_sc as plsc

NUM_TERMS = 16
NUM_COLS = 16384
DEPTH = 128
NUM_WORKERS = 32
COLS_PER_W = NUM_COLS // NUM_WORKERS
CHUNK = 128
NUM_CHUNKS = COLS_PER_W // CHUNK
LANES = 16
VECS_PER_ROW = DEPTH // LANES


def _body(x_hbm, pe_hbm, out_hbm,
          idx_v, acc0, acc1, acc2, acc3,
          gsem0, gsem1, gsem2, gsem3, wsem0):
    cid = lax.axis_index("c")
    sid = lax.axis_index("s")
    wid = sid * 2 + cid
    col0 = wid * COLS_PER_W
    inv = jnp.float32(1.0 / NUM_TERMS)
    zvec = jnp.zeros((LANES,), jnp.float32)

    accs = (acc0, acc1, acc2, acc3)
    gsems = (gsem0, gsem1, gsem2, gsem3)
    wsems = (wsem0,)

    def zero_acc(acc):
        def zbody(r, carry):
            for j in range(VECS_PER_ROW):
                acc[r, pl.ds(j * LANES, LANES)] = zvec
            return carry
        lax.fori_loop(0, CHUNK, zbody, 0)

    def fire(k):
        return [
            pltpu.async_copy(
                pe_hbm.at[idx_v.at[i, pl.ds(k * CHUNK, CHUNK)]],
                accs[k], gsems[k], add=True)
            for i in range(NUM_TERMS)
        ]

    idx0_cp = pltpu.async_copy(x_hbm.at[:, pl.ds(col0, CHUNK)],
                               idx_v.at[:, pl.ds(0, CHUNK)], wsem0)
    zero_acc(acc0)
    idx0_cp.wait()
    pending = {0: fire(0)}
    idxr_cp = pltpu.async_copy(
        x_hbm.at[:, pl.ds(col0 + CHUNK, COLS_PER_W - CHUNK)],
        idx_v.at[:, pl.ds(CHUNK, COLS_PER_W - CHUNK)], wsem0)
    zero_acc(acc1)
    idxr_cp.wait()
    pending[1] = fire(1)
    for k in range(2, NUM_CHUNKS):
        zero_acc(accs[k])
        pending[k] = fire(k)

    wb = []
    for k in range(NUM_CHUNKS):
        acc = accs[k]
        for cd in pending.pop(k):
            cd.wait()

        def row_body(r2, carry):
            for r in (2 * r2, 2 * r2 + 1):
                for j in range(VECS_PER_ROW):
                    sl = pl.ds(j * LANES, LANES)
                    acc[r, sl] = acc[r, sl] * inv
            return carry

        lax.fori_loop(0, CHUNK // 2, row_body, 0)
        wb.append(pltpu.async_copy(
            acc, out_hbm.at[pl.ds(col0 + k * CHUNK, CHUNK)], wsems[0]))
    for cd in wb:
        cd.wait()


@jax.jit
def kernel(x, position_encoding):
    mesh = plsc.VectorSubcoreMesh(core_axis_name="c", subcore_axis_name="s")
    f = pl.kernel(
        _body,
        mesh=mesh,
        out_type=jax.ShapeDtypeStruct((NUM_COLS, DEPTH), jnp.float32),
        scratch_types=(
            [pltpu.VMEM((NUM_TERMS, COLS_PER_W), jnp.int32)]
            + [pltpu.VMEM((CHUNK, DEPTH), jnp.float32)] * 4
            + [pltpu.SemaphoreType.DMA] * 5
        ),
    )
    return f(x, position_encoding)
